# Initial kernel scaffold; baseline (speedup 1.0000x reference)
import jax
import jax.numpy as jnp
from jax.experimental import pallas as pl


def kernel(data, gate_weight, w1, w2, w3, lora_a1, lora_b1, lora_a3, lora_b3, lora_a2, lora_b2):
    raise NotImplementedError("write your pallas kernel here")


# speedup vs baseline: 5.0244x; 5.0244x over previous
import functools

import jax
import jax.numpy as jnp
from jax.experimental import pallas as pl

_SCALING = 32.0 / 16.0


def _ffn_body(E, R, x_ref, gt_ref, w1t_ref, w3t_ref, w2t_ref,
              a1t_ref, b1_ref, a3t_ref, b3_ref, a2t_ref, b2_ref,
              out_ref, logits_ref):
    x = x_ref[...]
    f32 = jnp.float32
    logits = jnp.dot(x, gt_ref[...], preferred_element_type=f32)
    logits_ref[...] = logits
    m = jnp.max(logits, axis=-1, keepdims=True)
    ids_e = jax.lax.broadcasted_iota(jnp.int32, logits.shape, 1)
    e = jnp.min(jnp.where(logits == m, ids_e, E), axis=-1, keepdims=True)
    ids = jax.lax.broadcasted_iota(jnp.int32, (x.shape[0], E * R), 1)
    mask = (ids // R == e).astype(f32)

    la1 = jnp.dot(x, a1t_ref[...], preferred_element_type=f32) * mask
    g = (jnp.dot(x, w1t_ref[...], preferred_element_type=f32)
         + jnp.dot(la1, b1_ref[...], preferred_element_type=f32))
    la3 = jnp.dot(x, a3t_ref[...], preferred_element_type=f32) * mask
    u = (jnp.dot(x, w3t_ref[...], preferred_element_type=f32)
         + jnp.dot(la3, b3_ref[...], preferred_element_type=f32))
    h = (g * jax.lax.logistic(g)) * u
    la2 = jnp.dot(h, a2t_ref[...], preferred_element_type=f32) * mask
    out_ref[...] = (jnp.dot(h, w2t_ref[...], preferred_element_type=f32)
                    + jnp.dot(la2, b2_ref[...], preferred_element_type=f32))


def kernel(data, gate_weight, w1, w2, w3,
           lora_a1, lora_b1, lora_a3, lora_b3, lora_a2, lora_b2):
    T, D = data.shape
    F = w1.shape[0]
    E, R, _ = lora_a1.shape
    s = _SCALING

    gt = gate_weight.T
    w1t, w3t = w1.T, w3.T
    w2t = w2.T
    a1t = lora_a1.reshape(E * R, D).T
    b1c = lora_b1.transpose(0, 2, 1).reshape(E * R, F) * s
    a3t = lora_a3.reshape(E * R, D).T
    b3c = lora_b3.transpose(0, 2, 1).reshape(E * R, F) * s
    a2t = lora_a2.reshape(E * R, F).T
    b2c = lora_b2.transpose(0, 2, 1).reshape(E * R, D) * s

    TB = 256
    grid = (T // TB,)
    tok = lambda i: (i, 0)
    rep = lambda i: (0, 0)

    out, logits = pl.pallas_call(
        functools.partial(_ffn_body, E, R),
        grid=grid,
        in_specs=[
            pl.BlockSpec((TB, D), tok),
            pl.BlockSpec((D, E), rep),
            pl.BlockSpec((D, F), rep),
            pl.BlockSpec((D, F), rep),
            pl.BlockSpec((F, D), rep),
            pl.BlockSpec((D, E * R), rep),
            pl.BlockSpec((E * R, F), rep),
            pl.BlockSpec((D, E * R), rep),
            pl.BlockSpec((E * R, F), rep),
            pl.BlockSpec((F, E * R), rep),
            pl.BlockSpec((E * R, D), rep),
        ],
        out_specs=[
            pl.BlockSpec((TB, D), tok),
            pl.BlockSpec((TB, E), tok),
        ],
        out_shape=[
            jax.ShapeDtypeStruct((T, D), data.dtype),
            jax.ShapeDtypeStruct((T, E), data.dtype),
        ],
    )(data, gt, w1t, w3t, w2t, a1t, b1c, a3t, b3c, a2t, b2c)
    return out, logits


# speedup vs baseline: 6.8547x; 1.3643x over previous
import functools

import jax
import jax.numpy as jnp
from jax.experimental import pallas as pl

_SCALING = 32.0 / 16.0


def _ffn_body(E, R, logits_ref, xb_ref, w1t_ref, w3t_ref, w2t_ref,
              a1t_ref, b1_ref, a3t_ref, b3_ref, a2t_ref, b2_ref,
              out_ref):
    f32 = jnp.float32
    bf16 = jnp.bfloat16
    xb = xb_ref[...]
    logits = logits_ref[...]
    m = jnp.max(logits, axis=-1, keepdims=True)
    ids_e = jax.lax.broadcasted_iota(jnp.int32, logits.shape, 1)
    e = jnp.min(jnp.where(logits == m, ids_e, E), axis=-1, keepdims=True)
    ids = jax.lax.broadcasted_iota(jnp.int32, (xb.shape[0], E * R), 1)
    mask = (ids // R == e).astype(f32)

    la1 = (jnp.dot(xb, a1t_ref[...], preferred_element_type=f32) * mask).astype(bf16)
    g = (jnp.dot(xb, w1t_ref[...], preferred_element_type=f32)
         + jnp.dot(la1, b1_ref[...], preferred_element_type=f32))
    la3 = (jnp.dot(xb, a3t_ref[...], preferred_element_type=f32) * mask).astype(bf16)
    u = (jnp.dot(xb, w3t_ref[...], preferred_element_type=f32)
         + jnp.dot(la3, b3_ref[...], preferred_element_type=f32))
    h = ((g * jax.lax.logistic(g)) * u).astype(bf16)
    la2 = (jnp.dot(h, a2t_ref[...], preferred_element_type=f32) * mask).astype(bf16)
    out_ref[...] = (jnp.dot(h, w2t_ref[...], preferred_element_type=f32)
                    + jnp.dot(la2, b2_ref[...], preferred_element_type=f32))


def kernel(data, gate_weight, w1, w2, w3,
           lora_a1, lora_b1, lora_a3, lora_b3, lora_a2, lora_b2):
    T, D = data.shape
    F = w1.shape[0]
    E, R, _ = lora_a1.shape
    s = _SCALING
    bf16 = jnp.bfloat16

    router_logits = data @ gate_weight.T

    w1t, w3t = w1.T.astype(bf16), w3.T.astype(bf16)
    w2t = w2.T.astype(bf16)
    a1t = lora_a1.reshape(E * R, D).T.astype(bf16)
    b1c = (lora_b1.transpose(0, 2, 1).reshape(E * R, F) * s).astype(bf16)
    a3t = lora_a3.reshape(E * R, D).T.astype(bf16)
    b3c = (lora_b3.transpose(0, 2, 1).reshape(E * R, F) * s).astype(bf16)
    a2t = lora_a2.reshape(E * R, F).T.astype(bf16)
    b2c = (lora_b2.transpose(0, 2, 1).reshape(E * R, D) * s).astype(bf16)
    data_b = data.astype(bf16)

    TB = 256
    grid = (T // TB,)
    tok = lambda i: (i, 0)
    rep = lambda i: (0, 0)

    out = pl.pallas_call(
        functools.partial(_ffn_body, E, R),
        grid=grid,
        in_specs=[
            pl.BlockSpec((TB, E), tok),
            pl.BlockSpec((TB, D), tok),
            pl.BlockSpec((D, F), rep),
            pl.BlockSpec((D, F), rep),
            pl.BlockSpec((F, D), rep),
            pl.BlockSpec((D, E * R), rep),
            pl.BlockSpec((E * R, F), rep),
            pl.BlockSpec((D, E * R), rep),
            pl.BlockSpec((E * R, F), rep),
            pl.BlockSpec((F, E * R), rep),
            pl.BlockSpec((E * R, D), rep),
        ],
        out_specs=pl.BlockSpec((TB, D), tok),
        out_shape=jax.ShapeDtypeStruct((T, D), data.dtype),
    )(router_logits, data_b, w1t, w3t, w2t, a1t, b1c, a3t, b3c, a2t, b2c)
    return out, router_logits
